# baseline (speedup 1.0000x reference)
import functools

import jax
import jax.numpy as jnp
from jax import lax
from jax.experimental import pallas as pl
from jax.experimental.pallas import tpu as pltpu
from jax.experimental.pallas import tpu_sc as plsc

NUM_ROWS = 3
DIM = 256

NUM_CORES = 2
NUM_SUBCORES = 16
NUM_WORKERS = NUM_CORES * NUM_SUBCORES
LANES = 16
SLOTS = DIM // LANES

N_TOTAL = 16 * 128 * 128
RPW = N_TOTAL // NUM_WORKERS
CHUNK = 64
NCHUNKS = RPW // CHUNK
NBUF = 4
NGROUPS = NCHUNKS // NBUF


@functools.partial(
    pl.kernel,
    out_type=jax.ShapeDtypeStruct((N_TOTAL * DIM,), jnp.float32),
    mesh=plsc.VectorSubcoreMesh(core_axis_name="c", subcore_axis_name="s"),
    compiler_params=pltpu.CompilerParams(needs_layout_passes=False),
    scratch_types=[
        pltpu.VMEM((RPW,), jnp.int32),
        pltpu.VMEM((NUM_ROWS * DIM,), jnp.float32),
        pltpu.VMEM((CHUNK * DIM,), jnp.float32),
        pltpu.VMEM((CHUNK * DIM,), jnp.float32),
        pltpu.VMEM((CHUNK * DIM,), jnp.float32),
        pltpu.VMEM((CHUNK * DIM,), jnp.float32),
        pltpu.SemaphoreType.DMA,
    ],
)
def _sc_expand(idx_hbm, table_hbm, out_hbm, idx_v, table_v, rows0, rows1, rows2, rows3, ssem):
    wid = lax.axis_index("s") * NUM_CORES + lax.axis_index("c")
    base = wid * RPW
    pltpu.sync_copy(table_hbm, table_v)
    pltpu.sync_copy(idx_hbm.at[pl.ds(base, RPW)], idx_v)

    def bucket_body(i, carry):
        s = idx_v[pl.ds(i * LANES, LANES)]
        idx_v[pl.ds(i * LANES, LANES)] = jnp.clip(s - 1, 0, NUM_ROWS - 1) * DIM
        return carry

    lax.fori_loop(0, RPW // LANES, bucket_body, 0)

    iota = lax.iota(jnp.int32, LANES)
    slot_off = [iota + (s * LANES) for s in range(SLOTS)]

    def expand_chunk(c, buf):
        @plsc.parallel_loop(c * CHUNK, (c + 1) * CHUNK, step=1, unroll=8)
        def row_body(r):
            bvec = plsc.load_gather(idx_v, [jnp.full((LANES,), r, jnp.int32)])
            rbase = (r - c * CHUNK) * DIM
            for s in range(SLOTS):
                vals = plsc.load_gather(table_v, [bvec + slot_off[s]])
                buf[pl.ds(rbase + s * LANES, LANES)] = vals

    def start_scatter(c, buf):
        return pltpu.async_copy(
            buf, out_hbm.at[pl.ds((base + c * CHUNK) * DIM, CHUNK * DIM)], ssem
        )

    def drain_one(buf):
        pltpu.make_async_copy(
            buf, out_hbm.at[pl.ds(base * DIM, CHUNK * DIM)], ssem
        ).wait()

    bufs = (rows0, rows1, rows2, rows3)

    def group_body(g, carry):
        for par, buf in enumerate(bufs):
            c = g * NBUF + par

            @pl.when(g > 0)
            def _():
                drain_one(buf)

            expand_chunk(c, buf)
            start_scatter(c, buf)
        return carry

    lax.fori_loop(0, NGROUPS, group_body, 0)
    for buf in bufs:
        drain_one(buf)


def kernel(path_length, bucket_embedding):
    shape = path_length.shape
    flat_idx = path_length.reshape(-1).astype(jnp.int32)
    out = _sc_expand(flat_idx, bucket_embedding.reshape(-1))
    return out.reshape(*shape, DIM)


# speedup vs baseline: 1.1928x; 1.1928x over previous
import functools

import jax
import jax.numpy as jnp
from jax import lax
from jax.experimental import pallas as pl
from jax.experimental.pallas import tpu as pltpu
from jax.experimental.pallas import tpu_sc as plsc

NUM_ROWS = 3
DIM = 256

NUM_CORES = 2
NUM_SUBCORES = 16
NUM_WORKERS = NUM_CORES * NUM_SUBCORES
LANES = 16
SLOTS = DIM // LANES

N_TOTAL = 16 * 128 * 128
SC_ROWS = 7 * 8192
TC_ROWS = N_TOTAL - SC_ROWS

RPW = SC_ROWS // NUM_WORKERS
CHUNK = 128
NCHUNKS = RPW // CHUNK
NPAIRS = NCHUNKS // 2

TC_BLOCK = 2048
TC_GRID = TC_ROWS // TC_BLOCK


@functools.partial(
    pl.kernel,
    out_type=jax.ShapeDtypeStruct((SC_ROWS * DIM,), jnp.float32),
    mesh=plsc.VectorSubcoreMesh(core_axis_name="c", subcore_axis_name="s"),
    compiler_params=pltpu.CompilerParams(needs_layout_passes=False),
    scratch_types=[
        pltpu.VMEM((RPW,), jnp.int32),
        pltpu.VMEM((NUM_ROWS * DIM,), jnp.float32),
        pltpu.VMEM((CHUNK * DIM,), jnp.float32),
        pltpu.VMEM((CHUNK * DIM,), jnp.float32),
        pltpu.SemaphoreType.DMA,
    ],
)
def _sc_expand(idx_hbm, table_hbm, out_hbm, idx_v, table_v, rows0, rows1, ssem):
    wid = lax.axis_index("s") * NUM_CORES + lax.axis_index("c")
    base = wid * RPW
    pltpu.sync_copy(table_hbm, table_v)
    pltpu.sync_copy(idx_hbm.at[pl.ds(base, RPW)], idx_v)

    def bucket_body(i, carry):
        s = idx_v[pl.ds(i * LANES, LANES)]
        idx_v[pl.ds(i * LANES, LANES)] = jnp.clip(s - 1, 0, NUM_ROWS - 1) * DIM
        return carry

    lax.fori_loop(0, RPW // LANES, bucket_body, 0)

    iota = lax.iota(jnp.int32, LANES)
    slot_off = [iota + (s * LANES) for s in range(SLOTS)]

    def expand_chunk(c, buf):
        @plsc.parallel_loop(c * CHUNK, (c + 1) * CHUNK, step=1, unroll=4)
        def row_body(r):
            bvec = plsc.load_gather(idx_v, [jnp.full((LANES,), r, jnp.int32)])
            rbase = (r - c * CHUNK) * DIM
            for s in range(SLOTS):
                vals = plsc.load_gather(table_v, [bvec + slot_off[s]])
                buf[pl.ds(rbase + s * LANES, LANES)] = vals

    def start_scatter(c, buf):
        return pltpu.async_copy(
            buf, out_hbm.at[pl.ds((base + c * CHUNK) * DIM, CHUNK * DIM)], ssem
        )

    def drain_one(buf):
        pltpu.make_async_copy(
            buf, out_hbm.at[pl.ds(base * DIM, CHUNK * DIM)], ssem
        ).wait()

    def pair_body(g, carry):
        for par, buf in ((0, rows0), (1, rows1)):
            c = g * 2 + par

            @pl.when(g > 0)
            def _():
                drain_one(buf)

            expand_chunk(c, buf)
            start_scatter(c, buf)
        return carry

    lax.fori_loop(0, NPAIRS, pair_body, 0)
    drain_one(rows0)
    drain_one(rows1)


def _tc_body(idx_ref, table_ref, out_ref):
    idx = idx_ref[0, 0, :]
    b = jnp.clip(idx - 1, 0, NUM_ROWS - 1)
    b2 = b[:, None]
    row0 = table_ref[0:1, :]
    row1 = table_ref[1:2, :]
    row2 = table_ref[2:3, :]
    out_ref[0] = jnp.where(b2 == 0, row0, jnp.where(b2 == 1, row1, row2))


def _tc_expand(idx3, table):
    return pl.pallas_call(
        _tc_body,
        grid=(TC_GRID,),
        in_specs=[
            pl.BlockSpec((1, 1, TC_BLOCK), lambda i: (i, 0, 0)),
            pl.BlockSpec((NUM_ROWS, DIM), lambda i: (0, 0)),
        ],
        out_specs=pl.BlockSpec((1, TC_BLOCK, DIM), lambda i: (i, 0, 0)),
        out_shape=jax.ShapeDtypeStruct((TC_GRID, TC_BLOCK, DIM), jnp.float32),
    )(idx3, table)


def kernel(path_length, bucket_embedding):
    shape = path_length.shape
    flat = path_length.reshape(-1).astype(jnp.int32)
    sc_out = _sc_expand(flat[TC_ROWS:], bucket_embedding.reshape(-1))
    tc_out = _tc_expand(
        flat[:TC_ROWS].reshape(TC_GRID, 1, TC_BLOCK), bucket_embedding
    )
    out = jnp.concatenate(
        [tc_out.reshape(TC_ROWS, DIM), sc_out.reshape(SC_ROWS, DIM)], axis=0
    )
    return out.reshape(*shape, DIM)


# speedup vs baseline: 3.0856x; 2.5870x over previous
import functools

import jax
import jax.numpy as jnp
from jax import lax
from jax.experimental import pallas as pl
from jax.experimental.pallas import tpu as pltpu
from jax.experimental.pallas import tpu_sc as plsc

NUM_ROWS = 3
DIM = 256

NUM_CORES = 2
NUM_SUBCORES = 16
NUM_WORKERS = NUM_CORES * NUM_SUBCORES
LANES = 16
SLOTS = DIM // LANES

N_TOTAL = 16 * 128 * 128
SC_ROWS = 7 * 8192
TC_ROWS = N_TOTAL - SC_ROWS

RPW = SC_ROWS // NUM_WORKERS
CHUNK = 128
NCHUNKS = RPW // CHUNK
NPAIRS = NCHUNKS // 2

TC_BLOCK = 2048
TC_GRID = TC_ROWS // TC_BLOCK


@functools.partial(
    pl.kernel,
    out_type=jax.ShapeDtypeStruct((N_TOTAL, DIM), jnp.float32),
    mesh=plsc.VectorSubcoreMesh(core_axis_name="c", subcore_axis_name="s"),
    compiler_params=pltpu.CompilerParams(needs_layout_passes=False),
    scratch_types=[
        pltpu.VMEM((RPW,), jnp.int32),
        pltpu.VMEM((NUM_ROWS * DIM,), jnp.float32),
        pltpu.VMEM((CHUNK, DIM), jnp.float32),
        pltpu.VMEM((CHUNK, DIM), jnp.float32),
        pltpu.SemaphoreType.DMA,
    ],
)
def _sc_expand(idx_hbm, table_hbm, out_hbm, idx_v, table_v, rows0, rows1, ssem):
    wid = lax.axis_index("s") * NUM_CORES + lax.axis_index("c")
    base = wid * RPW
    pltpu.sync_copy(table_hbm, table_v)
    pltpu.sync_copy(idx_hbm.at[pl.ds(base, RPW)], idx_v)

    def bucket_body(i, carry):
        s = idx_v[pl.ds(i * LANES, LANES)]
        idx_v[pl.ds(i * LANES, LANES)] = jnp.clip(s - 1, 0, NUM_ROWS - 1) * DIM
        return carry

    lax.fori_loop(0, RPW // LANES, bucket_body, 0)

    iota = lax.iota(jnp.int32, LANES)
    slot_off = [iota + (s * LANES) for s in range(SLOTS)]

    def expand_chunk(c, buf):
        @plsc.parallel_loop(c * CHUNK, (c + 1) * CHUNK, step=1, unroll=4)
        def row_body(r):
            bvec = plsc.load_gather(idx_v, [jnp.full((LANES,), r, jnp.int32)])
            rloc = r - c * CHUNK
            for s in range(SLOTS):
                vals = plsc.load_gather(table_v, [bvec + slot_off[s]])
                buf[rloc, pl.ds(s * LANES, LANES)] = vals

    def start_scatter(c, buf):
        row = TC_ROWS + base + c * CHUNK
        return pltpu.async_copy(buf, out_hbm.at[pl.ds(row, CHUNK)], ssem)

    def drain_one(buf):
        pltpu.make_async_copy(
            buf, out_hbm.at[pl.ds(TC_ROWS + base, CHUNK)], ssem
        ).wait()

    def pair_body(g, carry):
        for par, buf in ((0, rows0), (1, rows1)):
            c = g * 2 + par

            @pl.when(g > 0)
            def _():
                drain_one(buf)

            expand_chunk(c, buf)
            start_scatter(c, buf)
        return carry

    lax.fori_loop(0, NPAIRS, pair_body, 0)
    drain_one(rows0)
    drain_one(rows1)


def _tc_body(idx_ref, table_ref, prev_ref, out_ref):
    del prev_ref
    idx = idx_ref[0, 0, :]
    b = jnp.clip(idx - 1, 0, NUM_ROWS - 1)
    b2 = b[:, None]
    row0 = table_ref[0:1, :]
    row1 = table_ref[1:2, :]
    row2 = table_ref[2:3, :]
    out_ref[...] = jnp.where(b2 == 0, row0, jnp.where(b2 == 1, row1, row2))


def _tc_expand(idx3, table, prev):
    return pl.pallas_call(
        _tc_body,
        grid=(TC_GRID,),
        in_specs=[
            pl.BlockSpec((1, 1, TC_BLOCK), lambda i: (i, 0, 0)),
            pl.BlockSpec((NUM_ROWS, DIM), lambda i: (0, 0)),
            pl.BlockSpec(memory_space=pl.ANY),
        ],
        out_specs=pl.BlockSpec((TC_BLOCK, DIM), lambda i: (i, 0)),
        out_shape=jax.ShapeDtypeStruct((N_TOTAL, DIM), jnp.float32),
        input_output_aliases={2: 0},
    )(idx3, table, prev)


def kernel(path_length, bucket_embedding):
    shape = path_length.shape
    flat = path_length.reshape(-1).astype(jnp.int32)
    sc_out = _sc_expand(flat[TC_ROWS:], bucket_embedding.reshape(-1))
    out = _tc_expand(
        flat[:TC_ROWS].reshape(TC_GRID, 1, TC_BLOCK), bucket_embedding, sc_out
    )
    return out.reshape(*shape, DIM)
